# Initial kernel scaffold; baseline (speedup 1.0000x reference)
import jax
import jax.numpy as jnp
from jax.experimental import pallas as pl


def kernel(cluster_assignments, generation, consumption, sharing_matrix, efficiency_matrix, weights):
    raise NotImplementedError("write your pallas kernel here")


# speedup vs baseline: 6.1812x; 6.1812x over previous
import functools

import jax
import jax.numpy as jnp
from jax.experimental import pallas as pl
from jax.experimental.pallas import tpu as pltpu

N = 4096
C = 64
BR = 256
CARBON_INTENSITY = 0.4


def _stream_body(seg_rows_ref, seg_full_ref, s_ref, e_ref,
                 colsum_ref, sent_ref, del_ref, colacc):
    i = pl.program_id(0)
    s = s_ref[0]
    e = e_ref[0]
    p = s * e
    psum = jnp.sum(p, axis=0)

    @pl.when(i == 0)
    def _():
        colacc[0, :] = psum

    @pl.when(i != 0)
    def _():
        colacc[0, :] = colacc[0, :] + psum

    seg_rows = seg_rows_ref[0]
    seg_full = seg_full_ref[0]
    mask = seg_rows[:, None] == seg_full[None, :]
    sent_ref[0, :] = jnp.sum(jnp.where(mask, s, 0.0), axis=1)
    del_ref[0, :] = jnp.sum(jnp.where(mask, p, 0.0), axis=1)

    @pl.when(i == (N // BR) - 1)
    def _():
        colsum_ref[0, :] = colacc[0, :]


def _metrics_body(seg_ref, gen_ref, cons_ref, w_ref,
                  colsum_ref, sent_ref, del_ref, out_ref):
    seg = seg_ref[0]
    gen = gen_ref[0]
    cons = cons_ref[0]
    colsum = colsum_ref[0]
    sent = sent_ref[0]
    dele = del_ref[0]

    cl = jax.lax.broadcasted_iota(jnp.int32, (C, N), 0)
    m = seg[None, :] == cl
    zeros = jnp.zeros((C, N), jnp.float32)
    neg = jnp.full((C, N), -jnp.inf, jnp.float32)

    count = jnp.sum(jnp.where(m, 1.0, 0.0), axis=1)
    total_gen = jnp.sum(jnp.where(m, gen[None, :], zeros), axis=1)
    total_cons = jnp.sum(jnp.where(m, cons[None, :], zeros), axis=1)
    sum_sq = jnp.sum(jnp.where(m, (cons * cons)[None, :], zeros), axis=1)
    peak_without = jnp.max(jnp.where(m, cons[None, :], neg), axis=1)
    net = cons - colsum
    peak_with = jnp.max(jnp.where(m, net[None, :], neg), axis=1)
    total_sent = jnp.sum(jnp.where(m, sent[None, :], zeros), axis=1)
    total_del = jnp.sum(jnp.where(m, dele[None, :], zeros), axis=1)

    local_energy_used = jnp.minimum(total_gen, total_cons)
    ssr = local_energy_used / (total_cons + 1e-06)
    peak_reduction = (peak_without - peak_with) / (peak_without + 1e-06)

    mean_c = total_cons / jnp.maximum(count, 1.0)
    var = (sum_sq - count * mean_c * mean_c) / jnp.maximum(count - 1.0, 1.0)
    std = jnp.sqrt(jnp.maximum(var, 1e-12))
    diversity_index = std / (mean_c + 1e-06)

    safe_sent = jnp.where(total_sent > 0, total_sent, 1.0)
    sharing_efficiency = jnp.where(total_sent > 0, total_del / safe_sent, 1.0)

    carbon_saved = local_energy_used * CARBON_INTENSITY

    w = w_ref[0]
    overall = (w[0] * ssr + w[1] * peak_reduction + w[2] * diversity_index +
               w[3] * sharing_efficiency + w[4] * (carbon_saved / 100.0))

    out_ref[...] = jnp.stack(
        [ssr, peak_reduction, diversity_index, sharing_efficiency,
         carbon_saved, overall], axis=1)


@jax.jit
def kernel(cluster_assignments, generation, consumption, sharing_matrix,
           efficiency_matrix, weights):
    seg = cluster_assignments.astype(jnp.int32)
    gen = generation
    cons = consumption
    w = weights.reshape(1, -1)

    grid = N // BR
    colsum, sent_row, del_row = pl.pallas_call(
        _stream_body,
        grid=(grid,),
        in_specs=[
            pl.BlockSpec((1, BR), lambda i: (0, i)),
            pl.BlockSpec((1, N), lambda i: (0, 0)),
            pl.BlockSpec((1, BR, N), lambda i: (0, i, 0)),
            pl.BlockSpec((1, BR, N), lambda i: (0, i, 0)),
        ],
        out_specs=[
            pl.BlockSpec((1, N), lambda i: (0, 0)),
            pl.BlockSpec((1, BR), lambda i: (0, i)),
            pl.BlockSpec((1, BR), lambda i: (0, i)),
        ],
        out_shape=[
            jax.ShapeDtypeStruct((1, N), jnp.float32),
            jax.ShapeDtypeStruct((1, N), jnp.float32),
            jax.ShapeDtypeStruct((1, N), jnp.float32),
        ],
        scratch_shapes=[pltpu.VMEM((1, N), jnp.float32)],
    )(seg, seg, sharing_matrix, efficiency_matrix)

    out = pl.pallas_call(
        _metrics_body,
        in_specs=[pl.BlockSpec(x.shape, lambda: (0, 0))
                  for x in (seg, gen, cons, w, colsum, sent_row, del_row)],
        out_specs=pl.BlockSpec((C, 6), lambda: (0, 0)),
        out_shape=jax.ShapeDtypeStruct((C, 6), jnp.float32),
    )(seg, gen, cons, w, colsum, sent_row, del_row)

    return out
